# baseline (speedup 1.0000x reference)
import functools

import jax
import jax.numpy as jnp
from jax import lax
from jax.experimental import pallas as pl
from jax.experimental.pallas import tpu as pltpu
from jax.experimental.pallas import tpu_sc as plsc

N = 10000
D = 128
E = 320000

NC = 2
NS = 16
NW = NC * NS
FPW = D // NW
WPW = FPW // 2
LANES = 16
CH = 3200
NCH = E // CH
GPC = CH // LANES
NEG1_PAIR = -1082081408


def _mlp1_body(x_ref, w_ref, b_ref, g_ref, be_ref, o_ref):
    h = jnp.dot(x_ref[...], w_ref[...], preferred_element_type=jnp.float32,
                precision=lax.Precision.HIGHEST)
    h = h + b_ref[...]
    mu = jnp.mean(h, axis=-1, keepdims=True)
    var = jnp.mean(jnp.square(h - mu), axis=-1, keepdims=True)
    h = (h - mu) / jnp.sqrt(var + 1e-5) * g_ref[...] + be_ref[...]
    o_ref[...] = jnp.maximum(h, 0.0)


def _mlp2_body(h_ref, a_ref, wa_ref, wb_ref, b_ref, g_ref, be_ref, o_ref):
    h = jnp.dot(h_ref[...], wa_ref[...], preferred_element_type=jnp.float32,
                precision=lax.Precision.HIGHEST)
    h = h + jnp.dot(a_ref[...], wb_ref[...], preferred_element_type=jnp.float32,
                    precision=lax.Precision.HIGHEST)
    h = h + b_ref[...]
    mu = jnp.mean(h, axis=-1, keepdims=True)
    var = jnp.mean(jnp.square(h - mu), axis=-1, keepdims=True)
    h = (h - mu) / jnp.sqrt(var + 1e-5) * g_ref[...] + be_ref[...]
    o_ref[...] = jnp.maximum(h, 0.0)


_ROWS = 1000


def _mlp1(x, w, b, g, be):
    vec = pl.BlockSpec((1, D), lambda i: (0, 0))
    return pl.pallas_call(
        _mlp1_body,
        grid=(N // _ROWS,),
        in_specs=[
            pl.BlockSpec((_ROWS, D), lambda i: (i, 0)),
            pl.BlockSpec((D, D), lambda i: (0, 0)),
            vec, vec, vec,
        ],
        out_specs=pl.BlockSpec((_ROWS, D), lambda i: (i, 0)),
        out_shape=jax.ShapeDtypeStruct((N, D), jnp.float32),
    )(x, w, b.reshape(1, D), g.reshape(1, D), be.reshape(1, D))


def _mlp2(h, a, wa, wb, b, g, be):
    vec = pl.BlockSpec((1, D), lambda i: (0, 0))
    return pl.pallas_call(
        _mlp2_body,
        grid=(N // _ROWS,),
        in_specs=[
            pl.BlockSpec((_ROWS, D), lambda i: (i, 0)),
            pl.BlockSpec((_ROWS, D), lambda i: (i, 0)),
            pl.BlockSpec((D, D), lambda i: (0, 0)),
            pl.BlockSpec((D, D), lambda i: (0, 0)),
            vec, vec, vec,
        ],
        out_specs=pl.BlockSpec((_ROWS, D), lambda i: (i, 0)),
        out_shape=jax.ShapeDtypeStruct((N, D), jnp.float32),
    )(h, a, wa, wb, b.reshape(1, D), g.reshape(1, D), be.reshape(1, D))


def _gcn_sc_body(hc, ep, out, hp0, hp1, ap0, ap1, scr0, scr1,
                 pbufa, pbufb, sema, semb):
    wid = lax.axis_index("s") * NC + lax.axis_index("c")
    lane = jnp.arange(LANES, dtype=jnp.int32)
    hps = (hp0, hp1)
    aps = (ap0, ap1)

    for j in range(WPW):
        pltpu.sync_copy(hc.at[wid, j], hps[j])

    def _init(i, carry):
        neg = jnp.full((LANES,), NEG1_PAIR, jnp.int32)
        for j in range(WPW):
            aps[j][pl.ds(i * LANES, LANES)] = neg
        return carry
    lax.fori_loop(0, N // LANES, _init, 0, unroll=8)

    def _start(c, pbuf, sem):
        pltpu.async_copy(ep.at[pl.ds(c * CH, CH)], pbuf, sem)

    def _wait(pbuf, sem):
        pltpu.make_async_copy(ep.at[pl.ds(0, CH)], pbuf, sem).wait()

    def _one_group(pbuf, g, scr):
        p = pbuf[pl.ds(g * LANES, LANES)]
        s = jnp.bitwise_and(p, jnp.int32(16383))
        d = jnp.right_shift(p, jnp.int32(14))
        plsc.store_scatter(scr, [d], lane)
        rb = plsc.load_gather(scr, [d])
        own = rb == lane
        vbs = []
        for j in range(WPW):
            pv = plsc.load_gather(hps[j], [s])
            cv = plsc.load_gather(aps[j], [d])
            vb = plsc.bitcast(pv, jnp.bfloat16)
            cb = plsc.bitcast(cv, jnp.bfloat16)
            mb = jnp.maximum(cb, vb)
            plsc.store_scatter(aps[j], [d], plsc.bitcast(mb, jnp.int32),
                               mask=own)
            vbs.append(vb)
        rem = jnp.logical_not(own)

        @pl.when(jnp.any(rem))
        def _fallback():
            def _cond(c2):
                r, it = c2
                return jnp.logical_and(jnp.any(r), it < LANES)

            def _body(c2):
                r, it = c2
                plsc.store_scatter(scr, [d], lane, mask=r)
                rb2 = plsc.load_gather(scr, [d])
                own2 = jnp.logical_and(rb2 == lane, r)
                for j in range(WPW):
                    cv2 = plsc.load_gather(aps[j], [d])
                    cb2 = plsc.bitcast(cv2, jnp.bfloat16)
                    mb2 = jnp.maximum(cb2, vbs[j])
                    plsc.store_scatter(aps[j], [d],
                                       plsc.bitcast(mb2, jnp.int32),
                                       mask=own2)
                return jnp.logical_and(r, jnp.logical_not(own2)), it + 1

            lax.while_loop(_cond, _body, (rem, jnp.int32(0)))

    def _groups(pbuf):
        def _pair(q, carry):
            _one_group(pbuf, q * 2, scr0)
            _one_group(pbuf, q * 2 + 1, scr1)
            return carry
        lax.fori_loop(0, GPC // 2, _pair, 0, unroll=2)

    _start(0, pbufa, sema)

    def _chunks(i, carry):
        ca = 2 * i
        _start(ca + 1, pbufb, semb)
        _wait(pbufa, sema)
        _groups(pbufa)

        @pl.when(ca + 2 < NCH)
        def _prefetch():
            _start(ca + 2, pbufa, sema)
        _wait(pbufb, semb)
        _groups(pbufb)
        return carry
    lax.fori_loop(0, NCH // 2, _chunks, 0)

    def _fix(i, carry):
        sl = pl.ds(i * LANES, LANES)
        for j in range(WPW):
            ab = plsc.bitcast(aps[j][sl], jnp.bfloat16)
            hb = plsc.bitcast(hps[j][sl], jnp.bfloat16)
            sel = jnp.where(ab < 0, hb, ab)
            aps[j][sl] = plsc.bitcast(sel, jnp.int32)
        return carry
    lax.fori_loop(0, N // LANES, _fix, 0, unroll=8)

    for j in range(WPW):
        pltpu.sync_copy(aps[j], out.at[wid, j])


@functools.partial(
    pl.kernel,
    mesh=plsc.VectorSubcoreMesh(core_axis_name="c", subcore_axis_name="s"),
    out_type=jax.ShapeDtypeStruct((NW, WPW, N), jnp.int32),
    compiler_params=pltpu.CompilerParams(needs_layout_passes=False),
    scratch_types=[
        pltpu.VMEM((N,), jnp.int32),
        pltpu.VMEM((N,), jnp.int32),
        pltpu.VMEM((N,), jnp.int32),
        pltpu.VMEM((N,), jnp.int32),
        pltpu.VMEM((N,), jnp.int32),
        pltpu.VMEM((N,), jnp.int32),
        pltpu.VMEM((CH,), jnp.int32),
        pltpu.VMEM((CH,), jnp.int32),
        pltpu.SemaphoreType.DMA,
        pltpu.SemaphoreType.DMA,
    ],
)
def _gcn_sc(hc, ep, out, hp0, hp1, ap0, ap1, scr0, scr1,
            pbufa, pbufb, sema, semb):
    _gcn_sc_body(hc, ep, out, hp0, hp1, ap0, ap1, scr0, scr1,
                 pbufa, pbufb, sema, semb)


def _to_packed(h):
    hb = h.astype(jnp.bfloat16).T
    hb = hb.reshape(NW, WPW, 2, N).transpose(0, 1, 3, 2)
    return lax.bitcast_convert_type(hb, jnp.int32)


def _from_packed(a):
    ab = lax.bitcast_convert_type(a, jnp.bfloat16)
    ab = ab.transpose(0, 1, 3, 2).reshape(D, N)
    return ab.T.astype(jnp.float32)


def kernel(x, edge_index, W0, b0, g0, be0, W1, b1, g1, be1):
    ei = edge_index.astype(jnp.int32)
    ep = ei[0] + ei[1] * jnp.int32(16384)

    h1 = _mlp1(x, W0, b0, g0, be0)
    a1 = _from_packed(_gcn_sc(_to_packed(h1), ep))
    h2 = _mlp2(h1, a1, W1[:D], W1[D:], b1, g1, be1)
    a2 = _from_packed(_gcn_sc(_to_packed(h2), ep))
    return jnp.concatenate([h2, a2], axis=1)
